# Initial kernel scaffold; baseline (speedup 1.0000x reference)
import jax
import jax.numpy as jnp
from jax.experimental import pallas as pl


def kernel(x, tables):
    raise NotImplementedError("write your pallas kernel here")


# speedup vs baseline: 1.2002x; 1.2002x over previous
import functools

import jax
import jax.numpy as jnp
from jax import lax
from jax.experimental import pallas as pl
from jax.experimental.pallas import tpu as pltpu
from jax.experimental.pallas import tpu_sc as plsc

N = 100000
C = 9
V = 200
D = 512
B = 8
G = B * C
NBLK = N // B
NW = 32
L = 16


@functools.partial(
    pl.kernel,
    out_type=jax.ShapeDtypeStruct((N, D), jnp.float32),
    mesh=plsc.VectorSubcoreMesh(core_axis_name="c", subcore_axis_name="s"),
    scratch_types=[
        pltpu.VMEM((G,), jnp.int32),
        pltpu.VMEM((G, D), jnp.float32),
        pltpu.VMEM((B, D), jnp.float32),
        pltpu.SemaphoreType.DMA,
    ],
)
def _atom_encoder_sc(idx_hbm, tabs_hbm, out_hbm, idx_v, rows_v, out_v, sem):
    w = lax.axis_index("s") * 2 + lax.axis_index("c")
    nblk_w = jnp.where(w < 20, 391, 390)
    blk0 = w * 390 + jnp.minimum(w, 20)

    def block_step(k, carry):
        blk = blk0 + k
        pltpu.sync_copy(idx_hbm.at[pl.ds(blk * G, G)], idx_v)
        pltpu.async_copy(tabs_hbm.at[idx_v], rows_v, sem).wait()

        def row_step(n, c1):
            def col_step(c, c2):
                sl = pl.ds(c * L, L)
                acc = rows_v[n * C + 0, sl]
                for i in range(1, C):
                    acc = acc + rows_v[n * C + i, sl]
                out_v[n, sl] = acc
                return c2

            return lax.fori_loop(0, D // L, col_step, c1)

        lax.fori_loop(0, B, row_step, 0)
        pltpu.sync_copy(out_v, out_hbm.at[pl.ds(blk * B, B)])
        return carry

    lax.fori_loop(0, nblk_w, block_step, 0)


def kernel(x, tables):
    offs = (jnp.arange(C, dtype=jnp.int32) * V)[None, :]
    idx = (x.astype(jnp.int32) + offs).reshape(N * C)
    tabs = tables.reshape(C * V, D)
    return _atom_encoder_sc(idx, tabs)


# speedup vs baseline: 1.9000x; 1.5830x over previous
import functools

import jax
import jax.numpy as jnp
from jax import lax
from jax.experimental import pallas as pl
from jax.experimental.pallas import tpu as pltpu
from jax.experimental.pallas import tpu_sc as plsc

N = 100000
C = 9
V = 200
D = 512
B = 8
G = B * C
NBLK = N // B
NW = 32
L = 16


@functools.partial(
    pl.kernel,
    out_type=jax.ShapeDtypeStruct((N, D), jnp.float32),
    mesh=plsc.VectorSubcoreMesh(core_axis_name="c", subcore_axis_name="s"),
    scratch_types=[
        pltpu.VMEM((2, G), jnp.int32),
        pltpu.VMEM((2, G, D), jnp.float32),
        pltpu.VMEM((B, D), jnp.float32),
        pltpu.SemaphoreType.DMA((2,)),
        pltpu.SemaphoreType.DMA((2,)),
    ],
)
def _atom_encoder_sc(idx_hbm, tabs_hbm, out_hbm, idx_v, rows_v, out_v,
                     sem_idx, sem_g):
    w = lax.axis_index("s") * 2 + lax.axis_index("c")
    nblk_w = jnp.where(w < 20, 391, 390)
    blk0 = w * 390 + jnp.minimum(w, 20)

    def idx_copy(blk, slot, sem):
        return pltpu.make_async_copy(
            idx_hbm.at[pl.ds(blk * G, G)], idx_v.at[slot], sem)

    def gather(slot, sem):
        return pltpu.make_async_copy(
            tabs_hbm.at[idx_v.at[slot]], rows_v.at[slot], sem)

    idx_copy(blk0, 0, sem_idx.at[0]).start()
    idx_copy(blk0, 0, sem_idx.at[0]).wait()
    gather(0, sem_g.at[0]).start()

    @pl.when(nblk_w > 1)
    def _():
        idx_copy(blk0 + 1, 1, sem_idx.at[1]).start()

    def block_step(k, carry):
        blk = blk0 + k
        buf = lax.rem(k, 2)
        nbuf = 1 - buf

        gather(buf, sem_g.at[buf]).wait()

        @pl.when(k + 2 < nblk_w)
        def _():
            idx_copy(blk + 2, buf, sem_idx.at[buf]).start()

        @pl.when(k + 1 < nblk_w)
        def _():
            idx_copy(blk + 1, nbuf, sem_idx.at[nbuf]).wait()
            gather(nbuf, sem_g.at[nbuf]).start()

        def row_step(n, c1):
            def col_step(c, c2):
                sl = pl.ds(c * L, L)
                acc = rows_v[buf, n * C + 0, sl]
                for i in range(1, C):
                    acc = acc + rows_v[buf, n * C + i, sl]
                out_v[n, sl] = acc
                return c2

            return lax.fori_loop(0, D // L, col_step, c1)

        lax.fori_loop(0, B, row_step, 0)
        pltpu.sync_copy(out_v, out_hbm.at[pl.ds(blk * B, B)])
        return carry

    lax.fori_loop(0, nblk_w, block_step, 0)


def kernel(x, tables):
    offs = (jnp.arange(C, dtype=jnp.int32) * V)[None, :]
    idx = (x.astype(jnp.int32) + offs).reshape(N * C)
    tabs = tables.reshape(C * V, D)
    return _atom_encoder_sc(idx, tabs)
